# baseline (speedup 1.0000x reference)
import functools

import jax
import jax.numpy as jnp
from jax import lax
from jax.experimental import pallas as pl
from jax.experimental.pallas import tpu as pltpu
from jax.experimental.pallas import tpu_sc as plsc

N = 10000
E = 320000
IN_DIM = 16
EMB = 128
ACT = 32

CHUNK = 128
NCHUNKS = 2512
EPAD = NCHUNKS * CHUNK
EDGE_BLK = 1256
NPER = NCHUNKS // 16
NSC = 2
NTILES = 16
ROWS_PER_TILE = 632
NPAD = ROWS_PER_TILE * NTILES
SUBCHUNKS = ((0, 128), (128, 128), (256, 128), (384, 128), (512, 120))


def _wfold_body(w2t, b2, cw1t0, cb10, cw1t1, cb11,
                m0_ref, beta0_ref, m1_ref, beta1_ref):
    dot = lambda a, b: jnp.dot(a, b, preferred_element_type=jnp.float32)
    m0_ref[...] = dot(w2t[...], cw1t0[...])
    beta0_ref[...] = dot(b2[...], cw1t0[...]) + cb10[...]
    m1_ref[...] = dot(w2t[...], cw1t1[...])
    beta1_ref[...] = dot(b2[...], cw1t1[...]) + cb11[...]


def _wfold_stage(w2t, b2, cw1t0, cb10, cw1t1, cb11):
    return pl.pallas_call(
        _wfold_body,
        out_shape=[jax.ShapeDtypeStruct((EMB, EMB), jnp.float32),
                   jax.ShapeDtypeStruct((1, EMB), jnp.float32),
                   jax.ShapeDtypeStruct((EMB, EMB), jnp.float32),
                   jax.ShapeDtypeStruct((1, EMB), jnp.float32)],
    )(w2t, b2, cw1t0, cb10, cw1t1, cb11)


def _edge_body(obs_ref, w1t, b1, m0, beta0, cw2t0, cb20,
               m1, beta1, cw2t1, cb21, a_ref):
    dot = lambda a, b: jnp.dot(a, b, preferred_element_type=jnp.float32)
    x = obs_ref[...]
    r = jnp.maximum(dot(x, w1t[...]) + b1[...], 0.0)
    a0 = jnp.maximum(dot(r, m0[...]) + beta0[...], 0.0)
    a_ref[0] = dot(a0, cw2t0[...]) + cb20[...]
    a1 = jnp.maximum(dot(r, m1[...]) + beta1[...], 0.0)
    a_ref[1] = dot(a1, cw2t1[...]) + cb21[...]


def _edge_stage(obs, *weights):
    nblk = EPAD // EDGE_BLK
    full = lambda shape: pl.BlockSpec(shape, lambda i: (0,) * len(shape))
    wspecs = [full((IN_DIM, EMB)), full((1, EMB))]
    for _ in range(4):
        wspecs += [full((EMB, EMB)), full((1, EMB))]
    return pl.pallas_call(
        _edge_body,
        grid=(nblk,),
        in_specs=[pl.BlockSpec((EDGE_BLK, IN_DIM), lambda i: (i, 0))] + wspecs,
        out_specs=pl.BlockSpec((2, EDGE_BLK, EMB), lambda i: (0, i, 0)),
        out_shape=jax.ShapeDtypeStruct((2, EPAD, EMB), jnp.float32),
    )(obs, *weights)


def _sc_body(a_hbm, dst_hbm, s_out,
             idx0, idx1, buf0, buf1, s_sh,
             sem_i0, sem_i1, sem_r0, sem_r1):
    c = lax.axis_index("c")
    s = lax.axis_index("s")
    r0 = pl.multiple_of(s * ROWS_PER_TILE, 8)
    idx_b = (idx0, idx1)
    bufs = (buf0, buf1)
    sem_i = (sem_i0, sem_i1)
    sem_r = (sem_r0, sem_r1)

    def _fill(i, _):
        zrow = jnp.zeros((16,), jnp.float32)
        def _z(j, _):
            buf0[i, pl.ds(j * 16, 16)] = zrow
            return 0
        lax.fori_loop(0, EMB // 16, _z, 0)
        return 0
    lax.fori_loop(0, CHUNK, _fill, 0)

    for off, sz in SUBCHUNKS:
        pltpu.sync_copy(buf0.at[pl.ds(0, sz)],
                        s_sh.at[pl.ds(r0 + off, sz)])

    plsc.subcore_barrier()

    def _load(j, b):
        chunk_id = s + j * NTILES
        pltpu.make_async_copy(dst_hbm.at[chunk_id], idx_b[b], sem_i[b]).start()
        pltpu.make_async_copy(a_hbm.at[c, chunk_id], bufs[b], sem_r[b]).start()

    def _wait_load(j, b):
        chunk_id = s + j * NTILES
        pltpu.make_async_copy(dst_hbm.at[chunk_id], idx_b[b], sem_i[b]).wait()
        pltpu.make_async_copy(a_hbm.at[c, chunk_id], bufs[b], sem_r[b]).wait()

    def _step(j, b):
        _load(j + 1, 1 - b)
        _wait_load(j, b)
        pltpu.sync_copy(bufs[b], s_sh.at[idx_b[b]], add=True)

    _load(0, 0)

    def _pair(t, _):
        _step(2 * t, 0)
        _step(2 * t + 1, 1)
        return 0

    lax.fori_loop(0, (NPER - 1) // 2, _pair, 0)
    _wait_load(NPER - 1, 0)
    pltpu.sync_copy(buf0, s_sh.at[idx0], add=True)

    plsc.subcore_barrier()

    for off, sz in SUBCHUNKS:
        pltpu.sync_copy(s_sh.at[pl.ds(r0 + off, sz)],
                        buf1.at[pl.ds(0, sz)])
        pltpu.sync_copy(buf1.at[pl.ds(0, sz)],
                        s_out.at[c, pl.ds(r0 + off, sz)])


def _sc_stage(a, dst2d):
    mesh = plsc.VectorSubcoreMesh(core_axis_name="c", subcore_axis_name="s")
    f = functools.partial(
        pl.kernel,
        out_type=jax.ShapeDtypeStruct((NSC, NPAD, EMB), jnp.float32),
        mesh=mesh,
        scratch_types=(
            [pltpu.VMEM((CHUNK,), jnp.int32)] * 2
            + [pltpu.VMEM((CHUNK, EMB), jnp.float32)] * 2
            + [pltpu.VMEM_SHARED((NPAD, EMB), jnp.float32)]
            + [pltpu.SemaphoreType.DMA] * 4
        ),
    )(_sc_body)
    return f(a, dst2d)


def _node_body(s_ref, cw3t0, cb30, cw4t0, cb40,
               cw3t1, cb31, cw4t1, cb41,
               w3t, b3, w4t, b4, out_ref):
    dot = lambda a, b: jnp.dot(a, b, preferred_element_type=jnp.float32)
    h = jnp.maximum(dot(s_ref[0], cw3t0[...]) + cb30[...], 0.0)
    h = jnp.maximum(dot(h, cw4t0[...]) + cb40[...], 0.0)
    hagg = s_ref[1] * h
    h = jnp.maximum(dot(hagg, cw3t1[...]) + cb31[...], 0.0)
    h = jnp.maximum(dot(h, cw4t1[...]) + cb41[...], 0.0)
    hg = jnp.sum(h, axis=0, keepdims=True)
    hg = jnp.maximum(dot(hg, w3t[...]) + b3[...], 0.0)
    out_ref[...] = dot(hg, w4t[...]) + b4[...]


def _node_stage(s, p):
    args = (s,
            p['c0_W3'].T, p['c0_b3'][None, :],
            p['c0_W4'].T, p['c0_b4'][None, :],
            p['c1_W3'].T, p['c1_b3'][None, :],
            p['c1_W4'].T, p['c1_b4'][None, :],
            p['W3'].T, p['b3'][None, :],
            p['W4'].T, p['b4'][None, :])
    return pl.pallas_call(
        _node_body,
        out_shape=jax.ShapeDtypeStruct((1, ACT), jnp.float32),
    )(*args)


def kernel(edge_index, obs, num_nodes, params):
    p = params
    del num_nodes
    dst2d = jnp.pad(edge_index[1].astype(jnp.int32).reshape(E // CHUNK, CHUNK),
                    ((0, NCHUNKS - E // CHUNK), (0, 0)), constant_values=N)

    m0, beta0, m1, beta1 = _wfold_stage(
        p['W2'].T, p['b2'][None, :],
        p['c0_W1'].T, p['c0_b1'][None, :],
        p['c1_W1'].T, p['c1_b1'][None, :])

    a = _edge_stage(jnp.pad(obs, ((0, EPAD - E), (0, 0))),
                    p['W1'].T, p['b1'][None, :],
                    m0, beta0, p['c0_W2'].T, p['c0_b2'][None, :],
                    m1, beta1, p['c1_W2'].T, p['c1_b2'][None, :])
    a4 = a.reshape(NSC, NCHUNKS, CHUNK, EMB)

    s = _sc_stage(a4, dst2d)

    return _node_stage(s[:, :N], p)


# speedup vs baseline: 1.3268x; 1.3268x over previous
import functools

import jax
import jax.numpy as jnp
from jax import lax
from jax.experimental import pallas as pl
from jax.experimental.pallas import tpu as pltpu
from jax.experimental.pallas import tpu_sc as plsc

N = 10000
E = 320000
IN_DIM = 16
EMB = 128
ACT = 32

CHUNK = 128
NCHUNKS = 2512
EPAD = NCHUNKS * CHUNK
EDGE_BLK = 5024
NPER = NCHUNKS // 16
NSC = 2
NTILES = 16
ROWS_PER_TILE = 632
NPAD = ROWS_PER_TILE * NTILES
SUBCHUNKS = ((0, 128), (128, 128), (256, 128), (384, 128), (512, 120))


def _wfold_body(w2t, b2, cw1t0, cb10, cw1t1, cb11,
                m0_ref, beta0_ref, m1_ref, beta1_ref):
    dot = lambda a, b: jnp.dot(a, b, preferred_element_type=jnp.float32)
    m0_ref[...] = dot(w2t[...], cw1t0[...])
    beta0_ref[...] = dot(b2[...], cw1t0[...]) + cb10[...]
    m1_ref[...] = dot(w2t[...], cw1t1[...])
    beta1_ref[...] = dot(b2[...], cw1t1[...]) + cb11[...]


def _wfold_stage(w2t, b2, cw1t0, cb10, cw1t1, cb11):
    return pl.pallas_call(
        _wfold_body,
        out_shape=[jax.ShapeDtypeStruct((EMB, EMB), jnp.float32),
                   jax.ShapeDtypeStruct((1, EMB), jnp.float32),
                   jax.ShapeDtypeStruct((EMB, EMB), jnp.float32),
                   jax.ShapeDtypeStruct((1, EMB), jnp.float32)],
    )(w2t, b2, cw1t0, cb10, cw1t1, cb11)


def _edge_body(obs_ref, w1t, b1, m0, beta0, cw2t0, cb20,
               m1, beta1, cw2t1, cb21, a_ref):
    dot = lambda a, b: jnp.dot(a, b, preferred_element_type=jnp.float32)
    x = obs_ref[...]
    r = jnp.maximum(dot(x, w1t[...]) + b1[...], 0.0)
    a0 = jnp.maximum(dot(r, m0[...]) + beta0[...], 0.0)
    a_ref[0] = dot(a0, cw2t0[...]) + cb20[...]
    a1 = jnp.maximum(dot(r, m1[...]) + beta1[...], 0.0)
    a_ref[1] = dot(a1, cw2t1[...]) + cb21[...]


def _edge_stage(obs, *weights):
    nblk = EPAD // EDGE_BLK
    full = lambda shape: pl.BlockSpec(shape, lambda i: (0,) * len(shape))
    wspecs = [full((IN_DIM, EMB)), full((1, EMB))]
    for _ in range(4):
        wspecs += [full((EMB, EMB)), full((1, EMB))]
    return pl.pallas_call(
        _edge_body,
        grid=(nblk,),
        in_specs=[pl.BlockSpec((EDGE_BLK, IN_DIM), lambda i: (i, 0))] + wspecs,
        out_specs=pl.BlockSpec((2, EDGE_BLK, EMB), lambda i: (0, i, 0)),
        out_shape=jax.ShapeDtypeStruct((2, EPAD, EMB), jnp.float32),
    )(obs, *weights)


def _sc_body(a_hbm, dst_hbm, s_out,
             idx0, idx1, buf0, buf1, s_sh,
             sem_i0, sem_i1, sem_r0, sem_r1):
    c = lax.axis_index("c")
    s = lax.axis_index("s")
    r0 = pl.multiple_of(s * ROWS_PER_TILE, 8)
    idx_b = (idx0, idx1)
    bufs = (buf0, buf1)
    sem_i = (sem_i0, sem_i1)
    sem_r = (sem_r0, sem_r1)

    def _fill(i, _):
        zrow = jnp.zeros((16,), jnp.float32)
        def _z(j, _):
            buf0[i, pl.ds(j * 16, 16)] = zrow
            return 0
        lax.fori_loop(0, EMB // 16, _z, 0)
        return 0
    lax.fori_loop(0, CHUNK, _fill, 0)

    for off, sz in SUBCHUNKS:
        pltpu.sync_copy(buf0.at[pl.ds(0, sz)],
                        s_sh.at[pl.ds(r0 + off, sz)])

    plsc.subcore_barrier()

    def _load(j, b):
        chunk_id = s + j * NTILES
        pltpu.make_async_copy(dst_hbm.at[chunk_id], idx_b[b], sem_i[b]).start()
        pltpu.make_async_copy(a_hbm.at[c, chunk_id], bufs[b], sem_r[b]).start()

    def _wait_load(j, b):
        chunk_id = s + j * NTILES
        pltpu.make_async_copy(dst_hbm.at[chunk_id], idx_b[b], sem_i[b]).wait()
        pltpu.make_async_copy(a_hbm.at[c, chunk_id], bufs[b], sem_r[b]).wait()

    def _step(j, b):
        _load(j + 1, 1 - b)
        _wait_load(j, b)
        pltpu.sync_copy(bufs[b], s_sh.at[idx_b[b]], add=True)

    _load(0, 0)

    def _pair(t, _):
        _step(2 * t, 0)
        _step(2 * t + 1, 1)
        return 0

    lax.fori_loop(0, (NPER - 1) // 2, _pair, 0)
    _wait_load(NPER - 1, 0)
    pltpu.sync_copy(buf0, s_sh.at[idx0], add=True)

    plsc.subcore_barrier()

    for off, sz in SUBCHUNKS:
        pltpu.sync_copy(s_sh.at[pl.ds(r0 + off, sz)],
                        buf1.at[pl.ds(0, sz)])
        pltpu.sync_copy(buf1.at[pl.ds(0, sz)],
                        s_out.at[c, pl.ds(r0 + off, sz)])


def _sc_stage(a, dst2d):
    mesh = plsc.VectorSubcoreMesh(core_axis_name="c", subcore_axis_name="s")
    f = functools.partial(
        pl.kernel,
        out_type=jax.ShapeDtypeStruct((NSC, NPAD, EMB), jnp.float32),
        mesh=mesh,
        scratch_types=(
            [pltpu.VMEM((CHUNK,), jnp.int32)] * 2
            + [pltpu.VMEM((CHUNK, EMB), jnp.float32)] * 2
            + [pltpu.VMEM_SHARED((NPAD, EMB), jnp.float32)]
            + [pltpu.SemaphoreType.DMA] * 4
        ),
    )(_sc_body)
    return f(a, dst2d)


def _node_body(s_ref, cw3t0, cb30, cw4t0, cb40,
               cw3t1, cb31, cw4t1, cb41,
               w3t, b3, w4t, b4, out_ref):
    dot = lambda a, b: jnp.dot(a, b, preferred_element_type=jnp.float32)
    h = jnp.maximum(dot(s_ref[0], cw3t0[...]) + cb30[...], 0.0)
    h = jnp.maximum(dot(h, cw4t0[...]) + cb40[...], 0.0)
    hagg = s_ref[1] * h
    h = jnp.maximum(dot(hagg, cw3t1[...]) + cb31[...], 0.0)
    h = jnp.maximum(dot(h, cw4t1[...]) + cb41[...], 0.0)
    hg = jnp.sum(h, axis=0, keepdims=True)
    hg = jnp.maximum(dot(hg, w3t[...]) + b3[...], 0.0)
    out_ref[...] = dot(hg, w4t[...]) + b4[...]


def _node_stage(s, p):
    args = (s,
            p['c0_W3'].T, p['c0_b3'][None, :],
            p['c0_W4'].T, p['c0_b4'][None, :],
            p['c1_W3'].T, p['c1_b3'][None, :],
            p['c1_W4'].T, p['c1_b4'][None, :],
            p['W3'].T, p['b3'][None, :],
            p['W4'].T, p['b4'][None, :])
    return pl.pallas_call(
        _node_body,
        out_shape=jax.ShapeDtypeStruct((1, ACT), jnp.float32),
    )(*args)


def kernel(edge_index, obs, num_nodes, params):
    p = params
    del num_nodes
    dst2d = jnp.pad(edge_index[1].astype(jnp.int32).reshape(E // CHUNK, CHUNK),
                    ((0, NCHUNKS - E // CHUNK), (0, 0)), constant_values=N)

    m0, beta0, m1, beta1 = _wfold_stage(
        p['W2'].T, p['b2'][None, :],
        p['c0_W1'].T, p['c0_b1'][None, :],
        p['c1_W1'].T, p['c1_b1'][None, :])

    a = _edge_stage(jnp.pad(obs, ((0, EPAD - E), (0, 0))),
                    p['W1'].T, p['b1'][None, :],
                    m0, beta0, p['c0_W2'].T, p['c0_b2'][None, :],
                    m1, beta1, p['c1_W2'].T, p['c1_b2'][None, :])
    a4 = a.reshape(NSC, NCHUNKS, CHUNK, EMB)

    s = _sc_stage(a4, dst2d)

    return _node_stage(s[:, :N], p)


# speedup vs baseline: 1.3690x; 1.0318x over previous
import functools

import jax
import jax.numpy as jnp
from jax import lax
from jax.experimental import pallas as pl
from jax.experimental.pallas import tpu as pltpu
from jax.experimental.pallas import tpu_sc as plsc

N = 10000
E = 320000
IN_DIM = 16
EMB = 128
ACT = 32

CHUNK = 128
NCHUNKS = 2512
EPAD = NCHUNKS * CHUNK
EDGE_BLK = 10048
NPER = NCHUNKS // 16
NSC = 2
NTILES = 16
ROWS_PER_TILE = 632
NPAD = ROWS_PER_TILE * NTILES
SUBCHUNKS = ((0, 128), (128, 128), (256, 128), (384, 128), (512, 120))


def _wfold_body(w2t, b2, cw1t0, cb10, cw1t1, cb11,
                m0_ref, beta0_ref, m1_ref, beta1_ref):
    dot = lambda a, b: jnp.dot(a, b, preferred_element_type=jnp.float32)
    m0_ref[...] = dot(w2t[...], cw1t0[...])
    beta0_ref[...] = dot(b2[...], cw1t0[...]) + cb10[...]
    m1_ref[...] = dot(w2t[...], cw1t1[...])
    beta1_ref[...] = dot(b2[...], cw1t1[...]) + cb11[...]


def _wfold_stage(w2t, b2, cw1t0, cb10, cw1t1, cb11):
    return pl.pallas_call(
        _wfold_body,
        out_shape=[jax.ShapeDtypeStruct((EMB, EMB), jnp.float32),
                   jax.ShapeDtypeStruct((1, EMB), jnp.float32),
                   jax.ShapeDtypeStruct((EMB, EMB), jnp.float32),
                   jax.ShapeDtypeStruct((1, EMB), jnp.float32)],
    )(w2t, b2, cw1t0, cb10, cw1t1, cb11)


def _edge_body(obs_ref, w1t, b1, m0, beta0, cw2t0, cb20,
               m1, beta1, cw2t1, cb21, a_ref):
    dot = lambda a, b: jnp.dot(a, b, preferred_element_type=jnp.float32)
    x = obs_ref[...]
    r = jnp.maximum(dot(x, w1t[...]) + b1[...], 0.0)
    a0 = jnp.maximum(dot(r, m0[...]) + beta0[...], 0.0)
    a_ref[0] = dot(a0, cw2t0[...]) + cb20[...]
    a1 = jnp.maximum(dot(r, m1[...]) + beta1[...], 0.0)
    a_ref[1] = dot(a1, cw2t1[...]) + cb21[...]


def _edge_stage(obs, *weights):
    nblk = EPAD // EDGE_BLK
    full = lambda shape: pl.BlockSpec(shape, lambda i: (0,) * len(shape))
    wspecs = [full((IN_DIM, EMB)), full((1, EMB))]
    for _ in range(4):
        wspecs += [full((EMB, EMB)), full((1, EMB))]
    return pl.pallas_call(
        _edge_body,
        grid=(nblk,),
        in_specs=[pl.BlockSpec((EDGE_BLK, IN_DIM), lambda i: (i, 0))] + wspecs,
        out_specs=pl.BlockSpec((2, EDGE_BLK, EMB), lambda i: (0, i, 0)),
        out_shape=jax.ShapeDtypeStruct((2, EPAD, EMB), jnp.float32),
    )(obs, *weights)


def _sc_body(a_hbm, dst_hbm, s_out,
             idx0, idx1, buf0, buf1, s_sh,
             sem_i0, sem_i1, sem_r0, sem_r1):
    c = lax.axis_index("c")
    s = lax.axis_index("s")
    r0 = pl.multiple_of(s * ROWS_PER_TILE, 8)
    idx_b = (idx0, idx1)
    bufs = (buf0, buf1)
    sem_i = (sem_i0, sem_i1)
    sem_r = (sem_r0, sem_r1)

    def _fill(i, _):
        zrow = jnp.zeros((16,), jnp.float32)
        def _z(j, _):
            buf0[i, pl.ds(j * 16, 16)] = zrow
            return 0
        lax.fori_loop(0, EMB // 16, _z, 0)
        return 0
    lax.fori_loop(0, CHUNK, _fill, 0)

    for off, sz in SUBCHUNKS:
        pltpu.sync_copy(buf0.at[pl.ds(0, sz)],
                        s_sh.at[pl.ds(r0 + off, sz)])

    plsc.subcore_barrier()

    def _load(j, b):
        chunk_id = s + j * NTILES
        pltpu.make_async_copy(dst_hbm.at[chunk_id], idx_b[b], sem_i[b]).start()
        pltpu.make_async_copy(a_hbm.at[c, chunk_id], bufs[b], sem_r[b]).start()

    def _wait_load(j, b):
        chunk_id = s + j * NTILES
        pltpu.make_async_copy(dst_hbm.at[chunk_id], idx_b[b], sem_i[b]).wait()
        pltpu.make_async_copy(a_hbm.at[c, chunk_id], bufs[b], sem_r[b]).wait()

    def _step(j, b):
        _load(j + 1, 1 - b)
        _wait_load(j, b)
        pltpu.sync_copy(bufs[b], s_sh.at[idx_b[b]], add=True)

    _load(0, 0)

    def _pair(t, _):
        _step(2 * t, 0)
        _step(2 * t + 1, 1)
        return 0

    lax.fori_loop(0, (NPER - 1) // 2, _pair, 0)
    _wait_load(NPER - 1, 0)
    pltpu.sync_copy(buf0, s_sh.at[idx0], add=True)

    plsc.subcore_barrier()

    for off, sz in SUBCHUNKS:
        pltpu.sync_copy(s_sh.at[pl.ds(r0 + off, sz)],
                        buf1.at[pl.ds(0, sz)])
        pltpu.sync_copy(buf1.at[pl.ds(0, sz)],
                        s_out.at[c, pl.ds(r0 + off, sz)])


def _sc_stage(a, dst2d):
    mesh = plsc.VectorSubcoreMesh(core_axis_name="c", subcore_axis_name="s")
    f = functools.partial(
        pl.kernel,
        out_type=jax.ShapeDtypeStruct((NSC, NPAD, EMB), jnp.float32),
        mesh=mesh,
        scratch_types=(
            [pltpu.VMEM((CHUNK,), jnp.int32)] * 2
            + [pltpu.VMEM((CHUNK, EMB), jnp.float32)] * 2
            + [pltpu.VMEM_SHARED((NPAD, EMB), jnp.float32)]
            + [pltpu.SemaphoreType.DMA] * 4
        ),
    )(_sc_body)
    return f(a, dst2d)


def _node_body(s_ref, cw3t0, cb30, cw4t0, cb40,
               cw3t1, cb31, cw4t1, cb41,
               w3t, b3, w4t, b4, out_ref):
    dot = lambda a, b: jnp.dot(a, b, preferred_element_type=jnp.float32)
    h = jnp.maximum(dot(s_ref[0], cw3t0[...]) + cb30[...], 0.0)
    h = jnp.maximum(dot(h, cw4t0[...]) + cb40[...], 0.0)
    hagg = s_ref[1] * h
    h = jnp.maximum(dot(hagg, cw3t1[...]) + cb31[...], 0.0)
    h = jnp.maximum(dot(h, cw4t1[...]) + cb41[...], 0.0)
    hg = jnp.sum(h, axis=0, keepdims=True)
    hg = jnp.maximum(dot(hg, w3t[...]) + b3[...], 0.0)
    out_ref[...] = dot(hg, w4t[...]) + b4[...]


def _node_stage(s, p):
    args = (s,
            p['c0_W3'].T, p['c0_b3'][None, :],
            p['c0_W4'].T, p['c0_b4'][None, :],
            p['c1_W3'].T, p['c1_b3'][None, :],
            p['c1_W4'].T, p['c1_b4'][None, :],
            p['W3'].T, p['b3'][None, :],
            p['W4'].T, p['b4'][None, :])
    return pl.pallas_call(
        _node_body,
        out_shape=jax.ShapeDtypeStruct((1, ACT), jnp.float32),
    )(*args)


def kernel(edge_index, obs, num_nodes, params):
    p = params
    del num_nodes
    dst2d = jnp.pad(edge_index[1].astype(jnp.int32).reshape(E // CHUNK, CHUNK),
                    ((0, NCHUNKS - E // CHUNK), (0, 0)), constant_values=N)

    m0, beta0, m1, beta1 = _wfold_stage(
        p['W2'].T, p['b2'][None, :],
        p['c0_W1'].T, p['c0_b1'][None, :],
        p['c1_W1'].T, p['c1_b1'][None, :])

    a = _edge_stage(jnp.pad(obs, ((0, EPAD - E), (0, 0))),
                    p['W1'].T, p['b1'][None, :],
                    m0, beta0, p['c0_W2'].T, p['c0_b2'][None, :],
                    m1, beta1, p['c1_W2'].T, p['c1_b2'][None, :])
    a4 = a.reshape(NSC, NCHUNKS, CHUNK, EMB)

    s = _sc_stage(a4, dst2d)

    return _node_stage(s[:, :N], p)
